# Initial kernel scaffold; baseline (speedup 1.0000x reference)
import jax
import jax.numpy as jnp
from jax.experimental import pallas as pl


def kernel(cached, bboxes, scores):
    raise NotImplementedError("write your pallas kernel here")


# speedup vs baseline: 460.2318x; 460.2318x over previous
import jax
import jax.numpy as jnp
from jax import lax
from jax.experimental import pallas as pl
from jax.experimental.pallas import tpu as pltpu
from jax.experimental.pallas import tpu_sc as plsc

NPROP = 300
N_TOTAL = 5300
L = 16
NCHUNK = (N_TOTAL + L - 1) // L
NPAD = NCHUNK * L
OUT_W = 5
OUT_PAD = 1504
SCORE_THR = 0.85
NMS_THR = 0.1
NEG = -3.0e38
DONE_THR = -1.0e37

_f32 = jnp.float32
_i32 = jnp.int32


def _nms_body(ms_h, x1_h, y1_h, x2_h, y2_h, out_h,
              s_v, x1_v, y1_v, x2_v, y2_v,
              cidx, cs, cx1, cy1, cx2, cy2, car, outf):
    cid = lax.axis_index("c")
    sid = lax.axis_index("s")

    @pl.when(sid == 0)
    def _run():
        lanes = lax.iota(_i32, L)
        zi = jnp.broadcast_to(_i32(0), (L,))
        negv = jnp.broadcast_to(_f32(NEG), (L,))
        zf = jnp.broadcast_to(_f32(0.0), (L,))
        bigi = _i32(2147483647)

        pltpu.sync_copy(ms_h, s_v)
        pltpu.sync_copy(x1_h, x1_v)
        pltpu.sync_copy(y1_h, y1_v)
        pltpu.sync_copy(x2_h, x2_v)
        pltpu.sync_copy(y2_h, y2_v)

        def _fill(i, _):
            cs[pl.ds(i * L, L)] = negv
            cidx[pl.ds(i * L, L)] = zi
            return 0

        lax.fori_loop(0, NCHUNK, _fill, 0)

        def _zout(i, _):
            outf[pl.ds(i * L, L)] = zf
            return 0

        lax.fori_loop(0, OUT_PAD // L, _zout, 0)

        onef = jnp.broadcast_to(_f32(1.0), (L,))

        def _compact(i, carry):
            off, vmax, vidx = carry
            s = s_v[pl.ds(i * L, L)]
            gidx = lanes + i * L
            m = s > _f32(SCORE_THR)
            mf = jnp.where(m, onef, zf)
            cum = plsc.cumsum(mf)
            dest = (cum + (off - _f32(1.0))).astype(_i32)
            plsc.store_scatter(cidx, [dest], gidx, mask=m)
            plsc.store_scatter(cs, [dest], s, mask=m)
            better = s > vmax
            vmax = jnp.where(better, s, vmax)
            vidx = jnp.where(better, gidx, vidx)
            return off + jnp.max(cum), vmax, vidx

        offf, vmax, vidx = lax.fori_loop(
            0, NCHUNK, _compact, (_f32(0.0), negv, zi))
        nv = offf.astype(_i32)

        @pl.when(nv == 0)
        def _fallback():
            mx = jnp.max(vmax)
            sel = jnp.min(jnp.where(vmax == mx, vidx, bigi))
            lane0 = lanes == 0
            plsc.store_scatter(cidx, [zi], jnp.broadcast_to(sel, (L,)),
                               mask=lane0)
            plsc.store_scatter(cs, [zi], jnp.broadcast_to(mx, (L,)),
                               mask=lane0)

        nv1 = jnp.maximum(nv, 1)
        nch = (nv1 + (L - 1)) // L

        csf = jnp.where(cid == 0, _f32(1.25), _f32(1.0))
        bsf = jnp.where(cid == 0, _f32(1.0), _f32(1.0 / 1.25))

        def _gather(i, _):
            idxv = cidx[pl.ds(i * L, L)]
            sf = jnp.where(idxv < NPROP, csf, bsf)
            x1 = plsc.load_gather(x1_v, [idxv]) * sf
            y1 = plsc.load_gather(y1_v, [idxv]) * sf
            x2 = plsc.load_gather(x2_v, [idxv]) * sf
            y2 = plsc.load_gather(y2_v, [idxv]) * sf
            ar = (jnp.maximum(x2 - x1, _f32(0.0))
                  * jnp.maximum(y2 - y1, _f32(0.0)))
            cx1[pl.ds(i * L, L)] = x1
            cy1[pl.ds(i * L, L)] = y1
            cx2[pl.ds(i * L, L)] = x2
            cy2[pl.ds(i * L, L)] = y2
            car[pl.ds(i * L, L)] = ar
            return 0

        lax.fori_loop(0, nch, _gather, 0)

        def _cond(st):
            k, done = st
            return (k < NPROP) & (done == 0)

        def _iter(st):
            k, done = st

            def _amax(i, c):
                bm, bi = c
                s = cs[pl.ds(i * L, L)]
                ci = lanes + i * L
                better = s > bm
                return jnp.where(better, s, bm), jnp.where(better, ci, bi)

            bm, bi = lax.fori_loop(0, nch, _amax, (negv, zi))
            mx = jnp.max(bm)
            sel = jnp.min(jnp.where(bm == mx, bi, bigi))
            found = mx > _f32(DONE_THR)

            @pl.when(found)
            def _emit():
                iv = jnp.broadcast_to(sel, (L,))
                x1s = plsc.load_gather(cx1, [iv])
                y1s = plsc.load_gather(cy1, [iv])
                x2s = plsc.load_gather(cx2, [iv])
                y2s = plsc.load_gather(cy2, [iv])
                ars = plsc.load_gather(car, [iv])
                mxv = jnp.broadcast_to(mx, (L,))
                row = jnp.where(lanes == 0, x1s,
                                jnp.where(lanes == 1, y1s,
                                          jnp.where(lanes == 2, x2s,
                                                    jnp.where(lanes == 3,
                                                              y2s, mxv))))
                oidx = lanes + k * OUT_W
                plsc.store_scatter(outf, [oidx], row, mask=lanes < OUT_W)

                def _suppress(i, _):
                    x1 = cx1[pl.ds(i * L, L)]
                    y1 = cy1[pl.ds(i * L, L)]
                    x2 = cx2[pl.ds(i * L, L)]
                    y2 = cy2[pl.ds(i * L, L)]
                    arc = car[pl.ds(i * L, L)]
                    xx1 = jnp.maximum(x1s, x1)
                    yy1 = jnp.maximum(y1s, y1)
                    xx2 = jnp.minimum(x2s, x2)
                    yy2 = jnp.minimum(y2s, y2)
                    inter = (jnp.maximum(xx2 - xx1, _f32(0.0))
                             * jnp.maximum(yy2 - yy1, _f32(0.0)))
                    denom = jnp.maximum(ars + arc - inter, _f32(1e-12))
                    iou = inter / denom
                    s = cs[pl.ds(i * L, L)]
                    cs[pl.ds(i * L, L)] = jnp.where(iou > _f32(NMS_THR),
                                                    negv, s)
                    return 0

                lax.fori_loop(0, nch, _suppress, 0)

            k = jnp.where(found, k + 1, k)
            done = jnp.where(found, _i32(0), _i32(1))
            return k, done

        lax.while_loop(_cond, _iter, (_i32(0), _i32(0)))

        pltpu.sync_copy(outf, out_h.at[cid])


def kernel(cached, bboxes, scores):
    cached = jnp.asarray(cached, _f32)
    bboxes = jnp.asarray(bboxes, _f32)
    scores = jnp.asarray(scores, _f32)

    pad = NPAD - N_TOTAL
    ms = jnp.concatenate([cached[:, 4], scores, jnp.full((pad,), NEG, _f32)])

    def col(j):
        return jnp.concatenate(
            [cached[:, j], bboxes[:, j], jnp.zeros((pad,), _f32)])

    mesh = plsc.VectorSubcoreMesh(core_axis_name="c", subcore_axis_name="s",
                                  num_cores=2, num_subcores=16)
    vec = lambda: pltpu.VMEM((NPAD,), _f32)
    out = pl.kernel(
        _nms_body,
        out_type=jax.ShapeDtypeStruct((2, OUT_PAD), _f32),
        mesh=mesh,
        compiler_params=pltpu.CompilerParams(needs_layout_passes=False),
        scratch_types=[
            vec(), vec(), vec(), vec(), vec(),
            pltpu.VMEM((NPAD,), _i32),
            vec(), vec(), vec(), vec(), vec(), vec(),
            pltpu.VMEM((OUT_PAD,), _f32),
        ],
    )(ms, col(0), col(1), col(2), col(3))
    return out[:, :NPROP * OUT_W].reshape(2, NPROP, OUT_W)


# speedup vs baseline: 550.7833x; 1.1968x over previous
import jax
import jax.numpy as jnp
from jax import lax
from jax.experimental import pallas as pl
from jax.experimental.pallas import tpu as pltpu
from jax.experimental.pallas import tpu_sc as plsc

NPROP = 300
N_TOTAL = 5300
L = 16
NCHUNK = (N_TOTAL + L - 1) // L
NPAD = NCHUNK * L
OUT_W = 5
OUT_PAD = 1504
SCORE_THR = 0.85
NMS_THR = 0.1
NEG = -3.0e38
DONE_THR = -1.0e37

_f32 = jnp.float32
_i32 = jnp.int32


def _nms_body(ms_h, x1_h, y1_h, x2_h, y2_h, out_h,
              s_v, x1_v, y1_v, x2_v, y2_v,
              cidx, cs, cx1, cy1, cx2, cy2, car, outf):
    cid = lax.axis_index("c")
    sid = lax.axis_index("s")

    @pl.when(sid == 0)
    def _run():
        lanes = lax.iota(_i32, L)
        zi = jnp.broadcast_to(_i32(0), (L,))
        negv = jnp.broadcast_to(_f32(NEG), (L,))
        zf = jnp.broadcast_to(_f32(0.0), (L,))
        bigi = _i32(2147483647)

        pltpu.sync_copy(ms_h, s_v)
        pltpu.sync_copy(x1_h, x1_v)
        pltpu.sync_copy(y1_h, y1_v)
        pltpu.sync_copy(x2_h, x2_v)
        pltpu.sync_copy(y2_h, y2_v)

        def _fill(i, _):
            cs[pl.ds(i * L, L)] = negv
            cidx[pl.ds(i * L, L)] = zi
            return 0

        lax.fori_loop(0, NCHUNK, _fill, 0)

        def _zout(i, _):
            outf[pl.ds(i * L, L)] = zf
            return 0

        lax.fori_loop(0, OUT_PAD // L, _zout, 0)

        onef = jnp.broadcast_to(_f32(1.0), (L,))

        def _compact(i, carry):
            off, vmax, vidx = carry
            s = s_v[pl.ds(i * L, L)]
            gidx = lanes + i * L
            m = s > _f32(SCORE_THR)
            mf = jnp.where(m, onef, zf)
            cum = plsc.cumsum(mf)
            dest = (cum + (off - _f32(1.0))).astype(_i32)
            plsc.store_scatter(cidx, [dest], gidx, mask=m)
            plsc.store_scatter(cs, [dest], s, mask=m)
            better = s > vmax
            vmax = jnp.where(better, s, vmax)
            vidx = jnp.where(better, gidx, vidx)
            return off + jnp.max(cum), vmax, vidx

        offf, vmax, vidx = lax.fori_loop(
            0, NCHUNK, _compact, (_f32(0.0), negv, zi))
        nv = offf.astype(_i32)

        @pl.when(nv == 0)
        def _fallback():
            mx = jnp.max(vmax)
            sel = jnp.min(jnp.where(vmax == mx, vidx, bigi))
            lane0 = lanes == 0
            plsc.store_scatter(cidx, [zi], jnp.broadcast_to(sel, (L,)),
                               mask=lane0)
            plsc.store_scatter(cs, [zi], jnp.broadcast_to(mx, (L,)),
                               mask=lane0)

        nv1 = jnp.maximum(nv, 1)
        nch = (nv1 + (L - 1)) // L

        csf = jnp.where(cid == 0, _f32(1.25), _f32(1.0))
        bsf = jnp.where(cid == 0, _f32(1.0), _f32(1.0 / 1.25))

        def _gather(i, c):
            bm, bi = c
            idxv = cidx[pl.ds(i * L, L)]
            sf = jnp.where(idxv < NPROP, csf, bsf)
            x1 = plsc.load_gather(x1_v, [idxv]) * sf
            y1 = plsc.load_gather(y1_v, [idxv]) * sf
            x2 = plsc.load_gather(x2_v, [idxv]) * sf
            y2 = plsc.load_gather(y2_v, [idxv]) * sf
            ar = (jnp.maximum(x2 - x1, _f32(0.0))
                  * jnp.maximum(y2 - y1, _f32(0.0)))
            cx1[pl.ds(i * L, L)] = x1
            cy1[pl.ds(i * L, L)] = y1
            cx2[pl.ds(i * L, L)] = x2
            cy2[pl.ds(i * L, L)] = y2
            car[pl.ds(i * L, L)] = ar
            s = cs[pl.ds(i * L, L)]
            ci = lanes + i * L
            better = s > bm
            return jnp.where(better, s, bm), jnp.where(better, ci, bi)

        bm0, bi0 = lax.fori_loop(0, nch, _gather, (negv, zi))
        mx0 = jnp.max(bm0)
        sel0 = jnp.min(jnp.where(bm0 == mx0, bi0, bigi))

        negone = jnp.broadcast_to(_i32(-1), (L,))

        def _cond(st):
            k, mx, sel, nc = st
            return (k < NPROP) & (mx > _f32(DONE_THR))

        def _iter(st):
            k, mx, sel, nc = st
            iv = jnp.broadcast_to(sel, (L,))
            x1s = plsc.load_gather(cx1, [iv])
            y1s = plsc.load_gather(cy1, [iv])
            x2s = plsc.load_gather(cx2, [iv])
            y2s = plsc.load_gather(cy2, [iv])
            ars = plsc.load_gather(car, [iv])
            mxv = jnp.broadcast_to(mx, (L,))
            row = jnp.where(lanes == 0, x1s,
                            jnp.where(lanes == 1, y1s,
                                      jnp.where(lanes == 2, x2s,
                                                jnp.where(lanes == 3,
                                                          y2s, mxv))))
            plsc.store_scatter(outf, [lanes + k * OUT_W], row,
                               mask=lanes < OUT_W)

            def _sweep(i, c):
                bm, bi, lastc = c
                x1 = cx1[pl.ds(i * L, L)]
                y1 = cy1[pl.ds(i * L, L)]
                x2 = cx2[pl.ds(i * L, L)]
                y2 = cy2[pl.ds(i * L, L)]
                arc = car[pl.ds(i * L, L)]
                xx1 = jnp.maximum(x1s, x1)
                yy1 = jnp.maximum(y1s, y1)
                xx2 = jnp.minimum(x2s, x2)
                yy2 = jnp.minimum(y2s, y2)
                inter = (jnp.maximum(xx2 - xx1, _f32(0.0))
                         * jnp.maximum(yy2 - yy1, _f32(0.0)))
                denom = jnp.maximum(ars + arc - inter, _f32(1e-12))
                iou = inter / denom
                s = cs[pl.ds(i * L, L)]
                s_new = jnp.where(iou > _f32(NMS_THR), negv, s)
                cs[pl.ds(i * L, L)] = s_new
                ci = lanes + i * L
                better = s_new > bm
                bm = jnp.where(better, s_new, bm)
                bi = jnp.where(better, ci, bi)
                alive = s_new > _f32(DONE_THR)
                lastc = jnp.where(alive, jnp.broadcast_to(i, (L,)), lastc)
                return bm, bi, lastc

            bm, bi, lastc = lax.fori_loop(0, nc, _sweep, (negv, zi, negone))
            mx2 = jnp.max(bm)
            sel2 = jnp.min(jnp.where(bm == mx2, bi, bigi))
            nc2 = jnp.max(lastc) + 1
            return k + 1, mx2, sel2, nc2

        lax.while_loop(_cond, _iter, (_i32(0), mx0, sel0, nch))

        pltpu.sync_copy(outf, out_h.at[cid])


def kernel(cached, bboxes, scores):
    cached = jnp.asarray(cached, _f32)
    bboxes = jnp.asarray(bboxes, _f32)
    scores = jnp.asarray(scores, _f32)

    pad = NPAD - N_TOTAL
    ms = jnp.concatenate([cached[:, 4], scores, jnp.full((pad,), NEG, _f32)])

    def col(j):
        return jnp.concatenate(
            [cached[:, j], bboxes[:, j], jnp.zeros((pad,), _f32)])

    mesh = plsc.VectorSubcoreMesh(core_axis_name="c", subcore_axis_name="s",
                                  num_cores=2, num_subcores=16)
    vec = lambda: pltpu.VMEM((NPAD,), _f32)
    out = pl.kernel(
        _nms_body,
        out_type=jax.ShapeDtypeStruct((2, OUT_PAD), _f32),
        mesh=mesh,
        compiler_params=pltpu.CompilerParams(needs_layout_passes=False),
        scratch_types=[
            vec(), vec(), vec(), vec(), vec(),
            pltpu.VMEM((NPAD,), _i32),
            vec(), vec(), vec(), vec(), vec(), vec(),
            pltpu.VMEM((OUT_PAD,), _f32),
        ],
    )(ms, col(0), col(1), col(2), col(3))
    return out[:, :NPROP * OUT_W].reshape(2, NPROP, OUT_W)


# speedup vs baseline: 1307.8889x; 2.3746x over previous
import jax
import jax.numpy as jnp
from jax import lax
from jax.experimental import pallas as pl
from jax.experimental.pallas import tpu as pltpu
from jax.experimental.pallas import tpu_sc as plsc

NPROP = 300
N_TOTAL = 5300
L = 16
NCHUNK = (N_TOTAL + L - 1) // L
NPAD = NCHUNK * L
OUT_W = 5
OUT_PAD = 1504
SCORE_THR = 0.85
NMS_THR = 0.1
NEG = -3.0e38
DONE_THR = -1.0e37

_f32 = jnp.float32
_i32 = jnp.int32


def _nms_body(ms_h, x1_h, y1_h, x2_h, y2_h, out_h,
              s_v, x1_v, y1_v, x2_v, y2_v,
              cidx, cs, cx1, cy1, cx2, cy2, car, outf):
    cid = lax.axis_index("c")
    sid = lax.axis_index("s")

    @pl.when(sid == 0)
    def _run():
        lanes = lax.iota(_i32, L)
        zi = jnp.broadcast_to(_i32(0), (L,))
        negv = jnp.broadcast_to(_f32(NEG), (L,))
        zf = jnp.broadcast_to(_f32(0.0), (L,))
        bigi = _i32(2147483647)

        pltpu.sync_copy(ms_h, s_v)
        pltpu.sync_copy(x1_h, x1_v)
        pltpu.sync_copy(y1_h, y1_v)
        pltpu.sync_copy(x2_h, x2_v)
        pltpu.sync_copy(y2_h, y2_v)

        @plsc.parallel_loop(0, NCHUNK, unroll=8)
        def _fill(i):
            cs[pl.ds(i * L, L)] = negv
            cidx[pl.ds(i * L, L)] = zi

        @plsc.parallel_loop(0, OUT_PAD // L, unroll=8)
        def _zout(i):
            outf[pl.ds(i * L, L)] = zf

        onef = jnp.broadcast_to(_f32(1.0), (L,))

        @plsc.parallel_loop(0, NCHUNK, unroll=4,
                            carry=(_f32(0.0), negv, zi))
        def _compact(i, carry):
            off, vmax, vidx = carry
            s = s_v[pl.ds(i * L, L)]
            gidx = lanes + i * L
            m = s > _f32(SCORE_THR)
            mf = jnp.where(m, onef, zf)
            cum = plsc.cumsum(mf)
            dest = (cum + (off - _f32(1.0))).astype(_i32)
            plsc.store_scatter(cidx, [dest], gidx, mask=m)
            plsc.store_scatter(cs, [dest], s, mask=m)
            better = s > vmax
            vmax = jnp.where(better, s, vmax)
            vidx = jnp.where(better, gidx, vidx)
            return off + jnp.max(cum), vmax, vidx

        offf, vmax, vidx = _compact
        nv = offf.astype(_i32)

        @pl.when(nv == 0)
        def _fallback():
            mx = jnp.max(vmax)
            sel = jnp.min(jnp.where(vmax == mx, vidx, bigi))
            lane0 = lanes == 0
            plsc.store_scatter(cidx, [zi], jnp.broadcast_to(sel, (L,)),
                               mask=lane0)
            plsc.store_scatter(cs, [zi], jnp.broadcast_to(mx, (L,)),
                               mask=lane0)

        nv1 = jnp.maximum(nv, 1)
        nch = (nv1 + (L - 1)) // L

        csf = jnp.where(cid == 0, _f32(1.25), _f32(1.0))
        bsf = jnp.where(cid == 0, _f32(1.0), _f32(1.0 / 1.25))

        @plsc.parallel_loop(0, nch, unroll=4, carry=(negv, zi))
        def _gather(i, c):
            bm, bi = c
            idxv = cidx[pl.ds(i * L, L)]
            sf = jnp.where(idxv < NPROP, csf, bsf)
            x1 = plsc.load_gather(x1_v, [idxv]) * sf
            y1 = plsc.load_gather(y1_v, [idxv]) * sf
            x2 = plsc.load_gather(x2_v, [idxv]) * sf
            y2 = plsc.load_gather(y2_v, [idxv]) * sf
            ar = (jnp.maximum(x2 - x1, _f32(0.0))
                  * jnp.maximum(y2 - y1, _f32(0.0)))
            cx1[pl.ds(i * L, L)] = x1
            cy1[pl.ds(i * L, L)] = y1
            cx2[pl.ds(i * L, L)] = x2
            cy2[pl.ds(i * L, L)] = y2
            car[pl.ds(i * L, L)] = ar
            s = cs[pl.ds(i * L, L)]
            ci = lanes + i * L
            better = s > bm
            return jnp.where(better, s, bm), jnp.where(better, ci, bi)

        bm0, bi0 = _gather
        mx0 = jnp.max(bm0)
        sel0 = jnp.min(jnp.where(bm0 == mx0, bi0, bigi))

        negone = jnp.broadcast_to(_i32(-1), (L,))

        def _cond(st):
            k, mx, sel, nc = st
            return (k < NPROP) & (mx > _f32(DONE_THR))

        def _iter(st):
            k, mx, sel, nc = st
            iv = jnp.broadcast_to(sel, (L,))
            x1s = plsc.load_gather(cx1, [iv])
            y1s = plsc.load_gather(cy1, [iv])
            x2s = plsc.load_gather(cx2, [iv])
            y2s = plsc.load_gather(cy2, [iv])
            ars = plsc.load_gather(car, [iv])
            mxv = jnp.broadcast_to(mx, (L,))
            row = jnp.where(lanes == 0, x1s,
                            jnp.where(lanes == 1, y1s,
                                      jnp.where(lanes == 2, x2s,
                                                jnp.where(lanes == 3,
                                                          y2s, mxv))))
            plsc.store_scatter(outf, [lanes + k * OUT_W], row,
                               mask=lanes < OUT_W)

            @plsc.parallel_loop(0, nc, unroll=4,
                                carry=(negv, zi, negone))
            def _sweep(i, c):
                bm, bi, lastc = c
                x1 = cx1[pl.ds(i * L, L)]
                y1 = cy1[pl.ds(i * L, L)]
                x2 = cx2[pl.ds(i * L, L)]
                y2 = cy2[pl.ds(i * L, L)]
                arc = car[pl.ds(i * L, L)]
                xx1 = jnp.maximum(x1s, x1)
                yy1 = jnp.maximum(y1s, y1)
                xx2 = jnp.minimum(x2s, x2)
                yy2 = jnp.minimum(y2s, y2)
                inter = (jnp.maximum(xx2 - xx1, _f32(0.0))
                         * jnp.maximum(yy2 - yy1, _f32(0.0)))
                denom = jnp.maximum(ars + arc - inter, _f32(1e-12))
                iou = inter / denom
                s = cs[pl.ds(i * L, L)]
                s_new = jnp.where(iou > _f32(NMS_THR), negv, s)
                cs[pl.ds(i * L, L)] = s_new
                ci = lanes + i * L
                better = s_new > bm
                bm = jnp.where(better, s_new, bm)
                bi = jnp.where(better, ci, bi)
                alive = s_new > _f32(DONE_THR)
                lastc = jnp.where(alive, jnp.broadcast_to(i, (L,)), lastc)
                return bm, bi, lastc

            bm, bi, lastc = _sweep
            mx2 = jnp.max(bm)
            sel2 = jnp.min(jnp.where(bm == mx2, bi, bigi))
            nc2 = jnp.max(lastc) + 1
            return k + 1, mx2, sel2, nc2

        lax.while_loop(_cond, _iter, (_i32(0), mx0, sel0, nch))

        pltpu.sync_copy(outf, out_h.at[cid])


def kernel(cached, bboxes, scores):
    cached = jnp.asarray(cached, _f32)
    bboxes = jnp.asarray(bboxes, _f32)
    scores = jnp.asarray(scores, _f32)

    pad = NPAD - N_TOTAL
    ms = jnp.concatenate([cached[:, 4], scores, jnp.full((pad,), NEG, _f32)])

    def col(j):
        return jnp.concatenate(
            [cached[:, j], bboxes[:, j], jnp.zeros((pad,), _f32)])

    mesh = plsc.VectorSubcoreMesh(core_axis_name="c", subcore_axis_name="s",
                                  num_cores=2, num_subcores=16)
    vec = lambda: pltpu.VMEM((NPAD,), _f32)
    out = pl.kernel(
        _nms_body,
        out_type=jax.ShapeDtypeStruct((2, OUT_PAD), _f32),
        mesh=mesh,
        compiler_params=pltpu.CompilerParams(needs_layout_passes=False),
        scratch_types=[
            vec(), vec(), vec(), vec(), vec(),
            pltpu.VMEM((NPAD,), _i32),
            vec(), vec(), vec(), vec(), vec(), vec(),
            pltpu.VMEM((OUT_PAD,), _f32),
        ],
    )(ms, col(0), col(1), col(2), col(3))
    return out[:, :NPROP * OUT_W].reshape(2, NPROP, OUT_W)
